# baseline (speedup 1.0000x reference)
import functools

import jax
import jax.numpy as jnp
from jax import lax
from jax.experimental import pallas as pl
from jax.experimental.pallas import tpu as pltpu
from jax.experimental.pallas import tpu_sc as plsc

VOCAB = 1000000
EMBED = 32
B, T = 4096, 200
N = B * T

_info = plsc.get_sparse_core_info()
NC, NS = _info.num_cores, _info.num_subcores
NW = NC * NS
PER_W = N // NW
GSIZE = 1280
NGROUP = PER_W // GSIZE

_mesh = plsc.VectorSubcoreMesh(core_axis_name="c", subcore_axis_name="s")


@functools.partial(
    pl.kernel,
    mesh=_mesh,
    out_type=jax.ShapeDtypeStruct((N, EMBED), jnp.float32),
    compiler_params=pltpu.CompilerParams(use_tc_tiling_on_sc=False),
    scratch_types=[
        pltpu.VMEM((PER_W,), jnp.int32),
        pltpu.VMEM((2, GSIZE, EMBED), jnp.float32),
        pltpu.SemaphoreType.DMA,
        pltpu.SemaphoreType.DMA,
        pltpu.SemaphoreType.DMA,
        pltpu.SemaphoreType.DMA,
    ],
)
def _embed_sc(idx_hbm, tab_hbm, out_hbm, idx_all, rows_v, sg0, sg1, ss0, ss1):
    wid = lax.axis_index("s") * NC + lax.axis_index("c")
    base = pl.multiple_of(wid * PER_W, GSIZE)
    sem_g = (sg0, sg1)
    sem_s = (ss0, ss1)

    pltpu.sync_copy(idx_hbm.at[pl.ds(base, PER_W)], idx_all)

    def fire(g, b):
        pltpu.async_copy(
            tab_hbm.at[idx_all.at[pl.ds(g * GSIZE, GSIZE)]],
            rows_v.at[b],
            sem_g[b],
        )

    def drain_gathers(b):
        pltpu.make_async_copy(
            out_hbm.at[pl.ds(0, GSIZE)], rows_v.at[b], sem_g[b]
        ).wait()

    def store(g, b):
        off = pl.multiple_of(base + g * GSIZE, GSIZE)
        pltpu.async_copy(rows_v.at[b], out_hbm.at[pl.ds(off, GSIZE)], sem_s[b])

    def drain_store(b):
        pltpu.make_async_copy(
            out_hbm.at[pl.ds(0, GSIZE)], rows_v.at[b], sem_s[b]
        ).wait()

    def step(g, b):
        drain_store(b)
        fire(g, b)
        drain_gathers(1 - b)
        store(g - 1, 1 - b)

    fire(0, 0)
    fire(1, 1)
    drain_gathers(0)
    store(0, 0)

    def body(k, carry):
        step(2 * k, 0)
        step(2 * k + 1, 1)
        return carry

    lax.fori_loop(1, NGROUP // 2, body, 0)

    drain_gathers(1)
    store(NGROUP - 1, 1)
    drain_store(0)
    drain_store(1)


def _fold_tc(x3):
    TB = 8
    UB = 4

    def body(in_ref, out_ref):
        x = in_ref[...]
        x5 = x.reshape(UB, 128, TB, EMBED // 8, 8)
        out_ref[...] = x5.transpose(2, 3, 0, 4, 1)

    return pl.pallas_call(
        body,
        grid=(T // TB, B // (UB * 128)),
        in_specs=[
            pl.BlockSpec((UB * 128, TB, EMBED), lambda t, ub: (ub, t, 0)),
        ],
        out_specs=pl.BlockSpec(
            (TB, EMBED // 8, UB, 8, 128), lambda t, ub: (t, 0, ub, 0, 0)
        ),
        out_shape=jax.ShapeDtypeStruct(
            (T, EMBED // 8, B // 128, 8, 128), jnp.float32
        ),
    )(x3)


def kernel(tokens, W_E):
    idx = tokens.reshape(N).astype(jnp.int32)
    out = _embed_sc(idx, W_E)
    out5 = _fold_tc(out.reshape(B, T, EMBED))
    return out5.transpose(2, 4, 0, 1, 3).reshape(B, T, EMBED)


# speedup vs baseline: 3.0909x; 3.0909x over previous
import functools

import jax
import jax.numpy as jnp
from jax import lax
from jax.experimental import pallas as pl
from jax.experimental.pallas import tpu as pltpu
from jax.experimental.pallas import tpu_sc as plsc

VOCAB = 1000000
EMBED = 32
B, T = 4096, 200
N = B * T

_info = plsc.get_sparse_core_info()
NC, NS = _info.num_cores, _info.num_subcores
NW = NC * NS
PER_W = N // NW
GSIZE = 1280
NGROUP = PER_W // GSIZE

_mesh = plsc.VectorSubcoreMesh(core_axis_name="c", subcore_axis_name="s")


@functools.partial(
    pl.kernel,
    mesh=_mesh,
    out_type=jax.ShapeDtypeStruct((N, EMBED), jnp.float32),
    compiler_params=pltpu.CompilerParams(use_tc_tiling_on_sc=False),
    scratch_types=[
        pltpu.VMEM((PER_W,), jnp.int32),
        pltpu.VMEM((2, GSIZE, EMBED), jnp.float32),
        pltpu.SemaphoreType.DMA,
        pltpu.SemaphoreType.DMA,
        pltpu.SemaphoreType.DMA,
        pltpu.SemaphoreType.DMA,
    ],
)
def _embed_sc(idx_hbm, tab_hbm, out_hbm, idx_all, rows_v, sg0, sg1, ss0, ss1):
    wid = lax.axis_index("s") * NC + lax.axis_index("c")
    base = pl.multiple_of(wid * PER_W, GSIZE)
    sem_g = (sg0, sg1)
    sem_s = (ss0, ss1)

    pltpu.sync_copy(idx_hbm.at[pl.ds(base, PER_W)], idx_all)

    def fire(g, b):
        pltpu.async_copy(
            tab_hbm.at[idx_all.at[pl.ds(g * GSIZE, GSIZE)]],
            rows_v.at[b],
            sem_g[b],
        )

    def drain_gathers(b):
        pltpu.make_async_copy(
            out_hbm.at[pl.ds(0, GSIZE)], rows_v.at[b], sem_g[b]
        ).wait()

    def store(g, b):
        off = pl.multiple_of(base + g * GSIZE, GSIZE)
        pltpu.async_copy(rows_v.at[b], out_hbm.at[pl.ds(off, GSIZE)], sem_s[b])

    def drain_store(b):
        pltpu.make_async_copy(
            out_hbm.at[pl.ds(0, GSIZE)], rows_v.at[b], sem_s[b]
        ).wait()

    def step(g, b):
        drain_store(b)
        fire(g, b)
        drain_gathers(1 - b)
        store(g - 1, 1 - b)

    fire(0, 0)
    fire(1, 1)
    drain_gathers(0)
    store(0, 0)

    def body(k, carry):
        step(2 * k, 0)
        step(2 * k + 1, 1)
        return carry

    lax.fori_loop(1, NGROUP // 2, body, 0)

    drain_gathers(1)
    store(NGROUP - 1, 1)
    drain_store(0)
    drain_store(1)


def kernel(tokens, W_E):
    idx = tokens.reshape(N).astype(jnp.int32)
    out = _embed_sc(idx, W_E)
    return out.reshape(B, T, EMBED)
